# Initial kernel scaffold; baseline (speedup 1.0000x reference)
import functools

import jax
import jax.numpy as jnp
from jax import lax
from jax.experimental import pallas as pl
from jax.experimental.pallas import tpu as pltpu
from jax.experimental.pallas import tpu_sc as plsc

VOCAB = 1000000
EMBED = 32
B = 16384 * 26

NUM_CORES = 2
NUM_SUBCORES = 16
NW = NUM_CORES * NUM_SUBCORES
B_PER_W = B // NW
CHUNK = 1024
NCHUNK = B_PER_W // CHUNK

_mesh = plsc.VectorSubcoreMesh(core_axis_name="c", subcore_axis_name="s")


@functools.partial(
    pl.kernel,
    mesh=_mesh,
    out_type=jax.ShapeDtypeStruct((B, EMBED), jnp.float32),
    scratch_types=[
        pltpu.VMEM((B_PER_W,), jnp.int32),
        pltpu.VMEM((CHUNK, EMBED), jnp.float32),
        pltpu.VMEM((CHUNK, EMBED), jnp.float32),
        pltpu.SemaphoreType.DMA,
        pltpu.SemaphoreType.DMA,
        pltpu.SemaphoreType.DMA,
        pltpu.SemaphoreType.DMA,
    ],
)
def _embed_gather(idx_hbm, table_hbm, out_hbm, idx_v, rows0, rows1,
                  gsem0, gsem1, osem0, osem1):
    wid = lax.axis_index("s") * NUM_CORES + lax.axis_index("c")
    base = wid * B_PER_W
    pltpu.sync_copy(idx_hbm.at[pl.ds(base, B_PER_W)], idx_v)

    bufs = (rows0, rows1)
    gsems = (gsem0, gsem1)
    osems = (osem0, osem1)

    def start_gather(c):
        return pltpu.async_copy(
            table_hbm.at[idx_v.at[pl.ds(c * CHUNK, CHUNK)]],
            bufs[c % 2], gsems[c % 2])

    def start_out(c):
        return pltpu.async_copy(
            bufs[c % 2], out_hbm.at[pl.ds(base + c * CHUNK, CHUNK)],
            osems[c % 2])

    gcp = start_gather(0)
    gnext = start_gather(1) if NCHUNK > 1 else None
    for c in range(NCHUNK):
        gcp.wait()
        ocp = start_out(c)
        ocp.wait()
        if c + 2 < NCHUNK:
            gcp, gnext = gnext, start_gather(c + 2)
        else:
            gcp = gnext


def kernel(x, weight):
    x_flat = x.reshape(-1).astype(jnp.int32)
    out = _embed_gather(x_flat, weight)
    return out.reshape(x.shape + (EMBED,))


# speedup vs baseline: 1.5753x; 1.5753x over previous
import functools

import jax
import jax.numpy as jnp
from jax import lax
from jax.experimental import pallas as pl
from jax.experimental.pallas import tpu as pltpu
from jax.experimental.pallas import tpu_sc as plsc

VOCAB = 1000000
EMBED = 32
B = 16384 * 26

NUM_CORES = 2
NUM_SUBCORES = 16
NW = NUM_CORES * NUM_SUBCORES
B_PER_W = B // NW
CHUNK = 1024
NCHUNK = B_PER_W // CHUNK

_mesh = plsc.VectorSubcoreMesh(core_axis_name="c", subcore_axis_name="s")


@functools.partial(
    pl.kernel,
    mesh=_mesh,
    out_type=jax.ShapeDtypeStruct((B, EMBED), jnp.float32),
    scratch_types=[
        pltpu.VMEM((B_PER_W,), jnp.int32),
        pltpu.VMEM((CHUNK, EMBED), jnp.float32),
        pltpu.VMEM((CHUNK, EMBED), jnp.float32),
        pltpu.SemaphoreType.DMA,
        pltpu.SemaphoreType.DMA,
        pltpu.SemaphoreType.DMA,
        pltpu.SemaphoreType.DMA,
    ],
    compiler_params=pltpu.CompilerParams(use_tc_tiling_on_sc=False),
)
def _embed_gather(idx_hbm, table_hbm, out_hbm, idx_v, rows0, rows1,
                  gsem0, gsem1, osem0, osem1):
    wid = lax.axis_index("s") * NUM_CORES + lax.axis_index("c")
    base = wid * B_PER_W
    pltpu.sync_copy(idx_hbm.at[pl.ds(base, B_PER_W)], idx_v)

    bufs = (rows0, rows1)
    gsems = (gsem0, gsem1)
    osems = (osem0, osem1)

    def start_gather(c):
        return pltpu.async_copy(
            table_hbm.at[idx_v.at[pl.ds(c * CHUNK, CHUNK)]],
            bufs[c % 2], gsems[c % 2])

    def start_out(c):
        return pltpu.async_copy(
            bufs[c % 2], out_hbm.at[pl.ds(base + c * CHUNK, CHUNK)],
            osems[c % 2])

    gcp = start_gather(0)
    gnext = start_gather(1) if NCHUNK > 1 else None
    for c in range(NCHUNK):
        gcp.wait()
        ocp = start_out(c)
        ocp.wait()
        if c + 2 < NCHUNK:
            gcp, gnext = gnext, start_gather(c + 2)
        else:
            gcp = gnext


def kernel(x, weight):
    x_flat = x.reshape(-1).astype(jnp.int32)
    out = _embed_gather(x_flat, weight)
    return out.reshape(x.shape + (EMBED,))


# speedup vs baseline: 1.7171x; 1.0900x over previous
import functools

import jax
import jax.numpy as jnp
from jax import lax
from jax.experimental import pallas as pl
from jax.experimental.pallas import tpu as pltpu
from jax.experimental.pallas import tpu_sc as plsc

VOCAB = 1000000
EMBED = 32
ROWS = 16384
COLS = 26
NUM_CORES = 2
NUM_SUBCORES = 16
NW = NUM_CORES * NUM_SUBCORES
R_PER_W = ROWS // NW

LINES = VOCAB // 4
CH_V = 1152
CH_L = CH_V // 4
N_FULL = 999936 // CH_V
TAIL_V = VOCAB - N_FULL * CH_V
TAIL_W = 4

_mesh = plsc.VectorSubcoreMesh(core_axis_name="c", subcore_axis_name="s")
_params = pltpu.CompilerParams(use_tc_tiling_on_sc=True,
                               needs_layout_passes=False)


def _wid():
    return lax.axis_index("s") * NUM_CORES + lax.axis_index("c")


def _iota():
    return jnp.arange(16, dtype=jnp.int32)


def _splat(x):
    return jnp.broadcast_to(jnp.asarray(x, jnp.int32), (16,))


@functools.partial(
    pl.kernel,
    mesh=_mesh,
    out_type=jax.ShapeDtypeStruct((LINES, 128), jnp.float32),
    scratch_types=[
        pltpu.VMEM((EMBED, CH_V), jnp.float32),
        pltpu.VMEM((EMBED, CH_V), jnp.float32),
        pltpu.VMEM((CH_L, 128), jnp.float32),
        pltpu.VMEM((TAIL_V, EMBED), jnp.float32),
        pltpu.SemaphoreType.DMA,
        pltpu.SemaphoreType.DMA,
    ],
    compiler_params=_params,
)
def _relayout(wt_hbm, tail_hbm, scr_hbm, src0, src1, dst_v, tail_v,
              isem0, isem1):
    wid = _wid()
    n_my = N_FULL // NW + jnp.where(wid < N_FULL % NW, 1, 0)

    e_vecs = (_iota(), _iota() + 16)
    lane_vecs = [_iota() + 16 * j for j in range(8)]

    def start_in(chunk, src_v, sem):
        return pltpu.async_copy(
            wt_hbm.at[:, pl.ds(chunk * CH_V, CH_V)], src_v, sem)

    def wait_in(chunk, src_v, sem):
        pltpu.make_async_copy(
            wt_hbm.at[:, pl.ds(chunk * CH_V, CH_V)], src_v, sem).wait()

    def do_chunk(chunk, src_v):
        @plsc.parallel_loop(0, CH_L, 1, unroll=8)
        def _(l):
            base = 4 * l
            v_vecs = [_splat(base + q) for q in range(4)]
            for j in range(8):
                vals = plsc.load_gather(src_v, [e_vecs[j & 1], v_vecs[j >> 1]])
                plsc.store_scatter(dst_v, [_splat(l), lane_vecs[j]], vals)

        pltpu.sync_copy(dst_v, scr_hbm.at[pl.ds(chunk * CH_L, CH_L)])

    start_in(wid, src0, isem0)

    def body(g2, carry):
        c0 = wid + (2 * g2) * NW
        c1 = c0 + NW
        start_in(c1, src1, isem1)
        wait_in(c0, src0, isem0)
        do_chunk(c0, src0)

        @pl.when(2 * g2 + 2 < n_my)
        def _():
            start_in(c1 + NW, src0, isem0)

        wait_in(c1, src1, isem1)
        do_chunk(c1, src1)

        @pl.when(2 * g2 + 3 < n_my)
        def _():
            start_in(c1 + 2 * NW, src1, isem1)

        return carry

    lax.fori_loop(0, n_my // 2, body, 0)

    @pl.when(n_my % 2 == 1)
    def _():
        c_last = wid + (n_my - 1) * NW
        wait_in(c_last, src0, isem0)
        do_chunk(c_last, src0)

    @pl.when(wid == TAIL_W)
    def _():
        pltpu.sync_copy(tail_hbm, tail_v)
        for lt in range(TAIL_V // 4):
            for j in range(8):
                vals = plsc.load_gather(
                    tail_v, [_splat(4 * lt + (j >> 1)), e_vecs[j & 1]])
                plsc.store_scatter(dst_v, [_splat(lt), lane_vecs[j]], vals)
        pltpu.sync_copy(dst_v.at[pl.ds(0, TAIL_V // 4)],
                        scr_hbm.at[pl.ds(N_FULL * CH_L, TAIL_V // 4)])


HCH = 256


@functools.partial(
    pl.kernel,
    mesh=_mesh,
    out_type=jax.ShapeDtypeStruct((COLS, EMBED, ROWS), jnp.float32),
    scratch_types=[
        pltpu.VMEM((COLS, R_PER_W), jnp.int32),
        pltpu.VMEM((HCH,), jnp.int32),
        pltpu.VMEM((HCH,), jnp.int32),
        pltpu.VMEM((HCH,), jnp.int32),
        pltpu.VMEM((HCH,), jnp.int32),
        pltpu.VMEM((HCH, 128), jnp.float32),
        pltpu.VMEM((HCH, 128), jnp.float32),
        pltpu.VMEM((4, 8, HCH), jnp.float32),
        pltpu.VMEM((4, 8, HCH), jnp.float32),
        pltpu.SemaphoreType.DMA,
        pltpu.SemaphoreType.DMA,
        pltpu.SemaphoreType.DMA,
        pltpu.SemaphoreType.DMA,
    ],
    compiler_params=_params,
)
def _embed_gather(idxt_hbm, scr_hbm, out_hbm, idx_v, li0, q0, li1, q1,
                  lines0, lines1, stage0, stage1, gsem0, gsem1,
                  osem0, osem1):
    wid = _wid()
    r0 = wid * R_PER_W
    pltpu.sync_copy(idxt_hbm.at[:, pl.ds(r0, R_PER_W)], idx_v)

    tr_vecs = [_splat(tr) for tr in range(4)]
    s_vecs = [_splat(s) for s in range(8)]

    def prep(c, h, li_v, q_v):
        base = _splat(h * HCH)
        cc = _splat(c)

        @plsc.parallel_loop(0, HCH // 16, 1, unroll=4)
        def _(k):
            loc = 16 * k + _iota()
            v = plsc.load_gather(idx_v, [cc, base + loc])
            plsc.store_scatter(li_v, [loc], v >> 2)
            plsc.store_scatter(q_v, [loc], v & 3)

    def start_gather(li_v, lines_v, sem):
        return pltpu.async_copy(scr_hbm.at[li_v], lines_v, sem)

    def wait_gather(li_v, lines_v, sem):
        pltpu.make_async_copy(scr_hbm.at[li_v], lines_v, sem).wait()

    def select(q_v, lines_v, stage_v):
        @plsc.parallel_loop(0, HCH // 16, 1, unroll=2)
        def _(j2):
            loc = 16 * j2 + _iota()
            colb = plsc.load_gather(q_v, [loc]) * 32
            for e in range(EMBED):
                vals = plsc.load_gather(lines_v, [loc, colb + e])
                plsc.store_scatter(
                    stage_v, [tr_vecs[e >> 3], s_vecs[e & 7], loc], vals)

    def start_outs(c, h, stage_v, sem):
        for tr in range(4):
            pltpu.async_copy(
                stage_v.at[tr],
                out_hbm.at[c, pl.ds(8 * tr, 8), pl.ds(r0 + h * HCH, HCH)],
                sem)

    def wait_outs(c, h, stage_v, sem):
        for tr in range(4):
            pltpu.make_async_copy(
                stage_v.at[tr],
                out_hbm.at[c, pl.ds(8 * tr, 8), pl.ds(r0 + h * HCH, HCH)],
                sem).wait()

    prep(0, 0, li0, q0)
    start_gather(li0, lines0, gsem0)

    def body(c, carry):
        prep(c, 1, li1, q1)
        start_gather(li1, lines1, gsem1)

        @pl.when(c > 0)
        def _():
            wait_outs(c - 1, 0, stage0, osem0)

        wait_gather(li0, lines0, gsem0)
        select(q0, lines0, stage0)
        start_outs(c, 0, stage0, osem0)

        @pl.when(c + 1 < COLS)
        def _():
            prep(c + 1, 0, li0, q0)
            start_gather(li0, lines0, gsem0)

        @pl.when(c > 0)
        def _():
            wait_outs(c - 1, 1, stage1, osem1)

        wait_gather(li1, lines1, gsem1)
        select(q1, lines1, stage1)
        start_outs(c, 1, stage1, osem1)
        return carry

    lax.fori_loop(0, COLS, body, 0)
    wait_outs(COLS - 1, 0, stage0, osem0)
    wait_outs(COLS - 1, 1, stage1, osem1)


def kernel(x, weight):
    scr = _relayout(weight.T, weight[N_FULL * CH_V:])
    p = _embed_gather(x.T.astype(jnp.int32), scr)
    return p.transpose(2, 0, 1)
